# baseline (speedup 1.0000x reference)
import functools

import jax
import jax.numpy as jnp
from jax import lax
from jax.experimental import pallas as pl
from jax.experimental.pallas import tpu as pltpu
from jax.experimental.pallas import tpu_sc as plsc

N_NODES = 10000
D = 128

NC = 2
NS = 16
NW = NC * NS

C = 120
K = 84
NBUF = 3
NGRP = K // NBUF
ZROWS = 8
EDGES_PAD = NW * K * C

ACC_ROWS = N_NODES


@functools.cache
def _build_sc_message_pass():
    mesh = plsc.VectorSubcoreMesh(core_axis_name="c", subcore_axis_name="s")
    return pl.kernel(
        _sc_message_pass_body,
        out_type=jax.ShapeDtypeStruct((NC, N_NODES, D), jnp.float32),
        mesh=mesh,
        scratch_types=(
            [pltpu.VMEM((2 * NBUF, C), jnp.int32) for _ in range(2)]
            + [pltpu.VMEM((C, D), jnp.float32) for _ in range(NBUF)]
            + [pltpu.SemaphoreType.DMA((2,)),
               pltpu.SemaphoreType.DMA((NBUF,)),
               pltpu.SemaphoreType.DMA((NBUF,)),
               pltpu.VMEM_SHARED((ACC_ROWS, D), jnp.float32)]
        ),
    )


def _sc_message_pass_body(feat_hbm, idx_hbm, zero_hbm, out_hbm, *rest):
    slots = rest[:2]
    bufs = rest[2:2 + NBUF]
    isem, gsem, ssem, acc_sh = rest[2 + NBUF:]
    cid = lax.axis_index("c")
    sid = lax.axis_index("s")
    wid = cid * NS + sid

    span = ACC_ROWS // NS

    def start_idx(g, s):
        pltpu.async_copy(idx_hbm.at[wid, g], slots[s], isem.at[s])

    def wait_idx(g, s):
        pltpu.make_async_copy(idx_hbm.at[wid, g], slots[s], isem.at[s]).wait()

    def start_gather(sl, pos, b):
        pltpu.async_copy(feat_hbm.at[sl.at[2 * pos]], bufs[b], gsem.at[b])

    def wait_gather(sl, pos, b):
        pltpu.make_async_copy(feat_hbm.at[sl.at[2 * pos]], bufs[b],
                              gsem.at[b]).wait()

    def start_scatter(sl, pos, b):
        pltpu.async_copy(bufs[b], acc_sh.at[sl.at[2 * pos + 1]], ssem.at[b],
                         add=True)

    def wait_scatter(sl, pos, b):
        pltpu.make_async_copy(bufs[b], acc_sh.at[sl.at[2 * pos + 1]],
                              ssem.at[b]).wait()

    def process_group(g, s, nxt, refill):
        sl, ns = slots[s], slots[1 - s]
        for b in range(NBUF):
            wait_gather(sl, b, b)
            start_scatter(sl, b, b)
        if nxt:
            wait_idx(g + 1, 1 - s)
        for b in range(NBUF):
            wait_scatter(sl, b, b)
            if nxt:
                start_gather(ns, b, b)
        if refill:
            start_idx(g + 2, s)

    start_idx(0, 0)
    start_idx(1, 1)
    wait_idx(0, 0)
    for b in range(NBUF):
        start_gather(slots[0], b, b)

    pltpu.sync_copy(zero_hbm, acc_sh.at[pl.ds(sid * span, span)])
    plsc.subcore_barrier()

    @pl.loop(0, NGRP - 4, step=2)
    def _(gp):
        process_group(gp, 0, True, True)
        process_group(gp + 1, 1, True, True)

    for g in range(NGRP - 4, NGRP):
        process_group(g, g % 2, g < NGRP - 1, g < NGRP - 2)

    plsc.subcore_barrier()

    rows_per = 624
    pltpu.sync_copy(acc_sh.at[pl.ds(sid * rows_per, rows_per)],
                    out_hbm.at[cid, pl.ds(sid * rows_per, rows_per)])

    @pl.when(sid == 0)
    def _():
        pltpu.sync_copy(acc_sh.at[pl.ds(NS * rows_per, N_NODES - NS * rows_per)],
                        out_hbm.at[cid, pl.ds(NS * rows_per, N_NODES - NS * rows_per)])


def _tc_linear_body(p_ref, w_ref, b_ref, o_ref):
    h = p_ref[0] + p_ref[1]
    y = lax.dot_general(
        h, w_ref[...],
        dimension_numbers=(((1,), (1,)), ((), ())),
        precision=lax.Precision.HIGHEST,
        preferred_element_type=jnp.float32,
    )
    o_ref[...] = jnp.maximum(y + b_ref[...], 0.0)


def kernel(feature, edge_index, W, b):
    n_edges = edge_index.shape[1]
    pad = EDGES_PAD - n_edges
    feat_aug = jnp.concatenate(
        [feature, jnp.zeros((ZROWS, D), feature.dtype)])
    pad_ar = jnp.arange(pad, dtype=jnp.int32)
    src = jnp.concatenate([edge_index[0], N_NODES + (pad_ar % ZROWS)])
    dst = jnp.concatenate([edge_index[1], pad_ar % N_NODES])
    idx = jnp.stack([src.reshape(NW, K, C), dst.reshape(NW, K, C)], axis=2)
    idx = idx.reshape(NW, NGRP, 2 * NBUF, C)

    zeros_blk = jnp.zeros((ACC_ROWS // NS, D), jnp.float32)
    partials = _build_sc_message_pass()(feat_aug, idx, zeros_blk)

    rows_blk = 1000
    grid = (N_NODES // rows_blk,)
    out = pl.pallas_call(
        _tc_linear_body,
        grid=grid,
        in_specs=[
            pl.BlockSpec((NC, rows_blk, D), lambda i: (0, i, 0)),
            pl.BlockSpec((D, D), lambda i: (0, 0)),
            pl.BlockSpec((1, D), lambda i: (0, 0)),
        ],
        out_specs=pl.BlockSpec((rows_blk, D), lambda i: (i, 0)),
        out_shape=jax.ShapeDtypeStruct((N_NODES, D), jnp.float32),
    )(partials, W, b.reshape(1, D))
    return out


# speedup vs baseline: 1.0321x; 1.0321x over previous
import functools

import jax
import jax.numpy as jnp
from jax import lax
from jax.experimental import pallas as pl
from jax.experimental.pallas import tpu as pltpu
from jax.experimental.pallas import tpu_sc as plsc

N_NODES = 10000
D = 128

NC = 2
NS = 16
NW = NC * NS

C = 120
K = 84
NBUF = 3
NGRP = K // NBUF
ZROWS = 8
EDGES_PAD = NW * K * C

ACC_ROWS = N_NODES


@functools.cache
def _build_sc_message_pass():
    mesh = plsc.VectorSubcoreMesh(core_axis_name="c", subcore_axis_name="s")
    return pl.kernel(
        _sc_message_pass_body,
        out_type=jax.ShapeDtypeStruct((NC, N_NODES, D), jnp.float32),
        mesh=mesh,
        scratch_types=(
            [pltpu.VMEM((NBUF, C), jnp.int32) for _ in range(4)]
            + [pltpu.VMEM((C, D), jnp.float32) for _ in range(NBUF)]
            + [pltpu.SemaphoreType.DMA((2,)),
               pltpu.SemaphoreType.DMA((NBUF,)),
               pltpu.SemaphoreType.DMA((NBUF,)),
               pltpu.VMEM_SHARED((ACC_ROWS, D), jnp.float32)]
        ),
    )


def _sc_message_pass_body(feat_hbm, src_hbm, dst_hbm, out_hbm, *rest):
    sslots = rest[:2]
    dslots = rest[2:4]
    bufs = rest[4:4 + NBUF]
    isem, gsem, ssem, acc_sh = rest[4 + NBUF:]
    cid = lax.axis_index("c")
    sid = lax.axis_index("s")
    wid = cid * NS + sid

    zbuf = bufs[0]

    @pl.loop(0, C)
    def _(r):
        @pl.loop(0, D, step=16)
        def _(c):
            zbuf.at[pl.ds(r, 1), pl.ds(c, 16)][...] = jnp.zeros(
                (1, 16), jnp.float32)

    span = ACC_ROWS // NS
    nfull = span // C
    for k in range(nfull):
        pltpu.sync_copy(zbuf, acc_sh.at[pl.ds(sid * span + k * C, C)])
    rem = span - nfull * C
    pltpu.sync_copy(zbuf.at[pl.ds(0, rem)],
                    acc_sh.at[pl.ds(sid * span + nfull * C, rem)])

    plsc.subcore_barrier()

    def start_idx(g, s):
        pltpu.async_copy(src_hbm.at[wid, g], sslots[s], isem.at[s])
        pltpu.async_copy(dst_hbm.at[wid, g], dslots[s], isem.at[s])

    def wait_idx(g, s):
        pltpu.make_async_copy(src_hbm.at[wid, g], sslots[s], isem.at[s]).wait()
        pltpu.make_async_copy(dst_hbm.at[wid, g], dslots[s], isem.at[s]).wait()

    def start_gather(sl, pos, b):
        pltpu.async_copy(feat_hbm.at[sl.at[pos]], bufs[b], gsem.at[b])

    def wait_gather(sl, pos, b):
        pltpu.make_async_copy(feat_hbm.at[sl.at[pos]], bufs[b],
                              gsem.at[b]).wait()

    def start_scatter(dl, pos, b):
        pltpu.async_copy(bufs[b], acc_sh.at[dl.at[pos]], ssem.at[b],
                         add=True)

    def wait_scatter(dl, pos, b):
        pltpu.make_async_copy(bufs[b], acc_sh.at[dl.at[pos]],
                              ssem.at[b]).wait()

    def process_group(g, s, nxt, refill):
        sl, dl, ns = sslots[s], dslots[s], sslots[1 - s]
        for b in range(NBUF):
            wait_gather(sl, b, b)
            start_scatter(dl, b, b)
        if nxt:
            wait_idx(g + 1, 1 - s)
        for b in range(NBUF):
            wait_scatter(dl, b, b)
            if nxt:
                start_gather(ns, b, b)
        if refill:
            start_idx(g + 2, s)

    start_idx(0, 0)
    start_idx(1, 1)
    wait_idx(0, 0)
    for b in range(NBUF):
        start_gather(sslots[0], b, b)

    @pl.loop(0, NGRP - 4, step=2)
    def _(gp):
        process_group(gp, 0, True, True)
        process_group(gp + 1, 1, True, True)

    for g in range(NGRP - 4, NGRP):
        process_group(g, g % 2, g < NGRP - 1, g < NGRP - 2)

    plsc.subcore_barrier()

    rows_per = 624
    pltpu.sync_copy(acc_sh.at[pl.ds(sid * rows_per, rows_per)],
                    out_hbm.at[cid, pl.ds(sid * rows_per, rows_per)])

    @pl.when(sid == 0)
    def _():
        pltpu.sync_copy(acc_sh.at[pl.ds(NS * rows_per, N_NODES - NS * rows_per)],
                        out_hbm.at[cid, pl.ds(NS * rows_per, N_NODES - NS * rows_per)])


def _tc_linear_body(p_ref, w_ref, b_ref, o_ref):
    h = p_ref[0] + p_ref[1]
    y = lax.dot_general(
        h, w_ref[...],
        dimension_numbers=(((1,), (1,)), ((), ())),
        preferred_element_type=jnp.float32,
    )
    o_ref[...] = jnp.maximum(y + b_ref[...], 0.0)


def kernel(feature, edge_index, W, b):
    n_edges = edge_index.shape[1]
    pad = EDGES_PAD - n_edges
    feat_aug = jnp.concatenate(
        [feature, jnp.zeros((ZROWS, D), feature.dtype)])
    pad_ar = jnp.arange(pad, dtype=jnp.int32)
    src = jnp.concatenate([edge_index[0], N_NODES + (pad_ar % ZROWS)])
    dst = jnp.concatenate([edge_index[1], pad_ar % N_NODES])
    src3 = src.reshape(NW, NGRP, NBUF, C)
    dst3 = dst.reshape(NW, NGRP, NBUF, C)

    partials = _build_sc_message_pass()(feat_aug, src3, dst3)

    rows_blk = 1000
    grid = (N_NODES // rows_blk,)
    out = pl.pallas_call(
        _tc_linear_body,
        grid=grid,
        in_specs=[
            pl.BlockSpec((NC, rows_blk, D), lambda i: (0, i, 0)),
            pl.BlockSpec((D, D), lambda i: (0, 0)),
            pl.BlockSpec((1, D), lambda i: (0, 0)),
        ],
        out_specs=pl.BlockSpec((rows_blk, D), lambda i: (i, 0)),
        out_shape=jax.ShapeDtypeStruct((N_NODES, D), jnp.float32),
    )(partials, W, b.reshape(1, D))
    return out


# speedup vs baseline: 1.0683x; 1.0351x over previous
import functools

import jax
import jax.numpy as jnp
from jax import lax
from jax.experimental import pallas as pl
from jax.experimental.pallas import tpu as pltpu
from jax.experimental.pallas import tpu_sc as plsc

N_NODES = 10000
D = 128

NC = 2
NS = 16
NW = NC * NS

C = 120
K = 84
NBUF = 3
NGRP = K // NBUF
EDGES_PAD = NW * K * C
PAD_EDGES = EDGES_PAD - 320000
PAD_ROWS = 16
PAD_PER_ROW = PAD_EDGES // PAD_ROWS

ACC_ROWS = N_NODES


@functools.cache
def _build_sc_message_pass():
    mesh = plsc.VectorSubcoreMesh(core_axis_name="c", subcore_axis_name="s")
    return pl.kernel(
        _sc_message_pass_body,
        out_type=jax.ShapeDtypeStruct((NC, N_NODES, D), jnp.float32),
        mesh=mesh,
        scratch_types=(
            [pltpu.VMEM((NBUF, C), jnp.int32) for _ in range(4)]
            + [pltpu.VMEM((C, D), jnp.float32) for _ in range(NBUF)]
            + [pltpu.SemaphoreType.DMA((2,)),
               pltpu.SemaphoreType.DMA((NBUF,)),
               pltpu.SemaphoreType.DMA((NBUF,)),
               pltpu.VMEM_SHARED((ACC_ROWS, D), jnp.float32)]
        ),
    )


def _sc_message_pass_body(feat_hbm, src_hbm, dst_hbm, out_hbm, *rest):
    sslots = rest[:2]
    dslots = rest[2:4]
    bufs = rest[4:4 + NBUF]
    isem, gsem, ssem, acc_sh = rest[4 + NBUF:]
    cid = lax.axis_index("c")
    sid = lax.axis_index("s")
    wid = cid * NS + sid

    zbuf = bufs[0]

    @pl.loop(0, C)
    def _(r):
        @pl.loop(0, D, step=16)
        def _(c):
            zbuf.at[pl.ds(r, 1), pl.ds(c, 16)][...] = jnp.zeros(
                (1, 16), jnp.float32)

    span = ACC_ROWS // NS
    nfull = span // C
    for k in range(nfull):
        pltpu.sync_copy(zbuf, acc_sh.at[pl.ds(sid * span + k * C, C)])
    rem = span - nfull * C
    pltpu.sync_copy(zbuf.at[pl.ds(0, rem)],
                    acc_sh.at[pl.ds(sid * span + nfull * C, rem)])

    plsc.subcore_barrier()

    def start_idx(g, s):
        pltpu.async_copy(src_hbm.at[wid, g], sslots[s], isem.at[s])
        pltpu.async_copy(dst_hbm.at[wid, g], dslots[s], isem.at[s])

    def wait_idx(g, s):
        pltpu.make_async_copy(src_hbm.at[wid, g], sslots[s], isem.at[s]).wait()
        pltpu.make_async_copy(dst_hbm.at[wid, g], dslots[s], isem.at[s]).wait()

    def start_gather(sl, pos, b):
        pltpu.async_copy(feat_hbm.at[sl.at[pos]], bufs[b], gsem.at[b])

    def wait_gather(sl, pos, b):
        pltpu.make_async_copy(feat_hbm.at[sl.at[pos]], bufs[b],
                              gsem.at[b]).wait()

    def start_scatter(dl, pos, b):
        pltpu.async_copy(bufs[b], acc_sh.at[dl.at[pos]], ssem.at[b],
                         add=True)

    def wait_scatter(dl, pos, b):
        pltpu.make_async_copy(bufs[b], acc_sh.at[dl.at[pos]],
                              ssem.at[b]).wait()

    def process_group(g, s, nxt, refill):
        sl, dl, ns = sslots[s], dslots[s], sslots[1 - s]
        for b in range(NBUF):
            wait_gather(sl, b, b)
            start_scatter(dl, b, b)
        if nxt:
            wait_idx(g + 1, 1 - s)
        for b in range(NBUF):
            wait_scatter(dl, b, b)
            if nxt:
                start_gather(ns, b, b)
        if refill:
            start_idx(g + 2, s)

    start_idx(0, 0)
    start_idx(1, 1)
    wait_idx(0, 0)
    for b in range(NBUF):
        start_gather(sslots[0], b, b)

    @pl.loop(0, NGRP - 4, step=2)
    def _(gp):
        process_group(gp, 0, True, True)
        process_group(gp + 1, 1, True, True)

    for g in range(NGRP - 4, NGRP):
        process_group(g, g % 2, g < NGRP - 1, g < NGRP - 2)

    plsc.subcore_barrier()

    rows_per = 624
    pltpu.sync_copy(acc_sh.at[pl.ds(sid * rows_per, rows_per)],
                    out_hbm.at[cid, pl.ds(sid * rows_per, rows_per)])

    @pl.when(sid == 0)
    def _():
        pltpu.sync_copy(acc_sh.at[pl.ds(NS * rows_per, N_NODES - NS * rows_per)],
                        out_hbm.at[cid, pl.ds(NS * rows_per, N_NODES - NS * rows_per)])


def _tc_linear_body(p_ref, ftop_ref, w_ref, b_ref, o_ref):
    h = p_ref[0] + p_ref[1]
    corr = jnp.concatenate(
        [ftop_ref[...] * float(PAD_PER_ROW),
         jnp.zeros((1000 - PAD_ROWS, D), jnp.float32)], axis=0)
    h = h - jnp.where(pl.program_id(0) == 0, 1.0, 0.0) * corr
    y = lax.dot_general(
        h, w_ref[...],
        dimension_numbers=(((1,), (1,)), ((), ())),
        preferred_element_type=jnp.float32,
    )
    o_ref[...] = jnp.maximum(y + b_ref[...], 0.0)


def kernel(feature, edge_index, W, b):
    n_edges = edge_index.shape[1]
    pad = EDGES_PAD - n_edges
    pad_idx = jnp.arange(pad, dtype=jnp.int32) % PAD_ROWS
    src = jnp.concatenate([edge_index[0], pad_idx])
    dst = jnp.concatenate([edge_index[1], pad_idx])
    src3 = src.reshape(NW, NGRP, NBUF, C)
    dst3 = dst.reshape(NW, NGRP, NBUF, C)

    partials = _build_sc_message_pass()(feature, src3, dst3)

    rows_blk = 1000
    grid = (N_NODES // rows_blk,)
    out = pl.pallas_call(
        _tc_linear_body,
        grid=grid,
        in_specs=[
            pl.BlockSpec((NC, rows_blk, D), lambda i: (0, i, 0)),
            pl.BlockSpec((PAD_ROWS, D), lambda i: (0, 0)),
            pl.BlockSpec((D, D), lambda i: (0, 0)),
            pl.BlockSpec((1, D), lambda i: (0, 0)),
        ],
        out_specs=pl.BlockSpec((rows_blk, D), lambda i: (i, 0)),
        out_shape=jax.ShapeDtypeStruct((N_NODES, D), jnp.float32),
    )(partials, feature[:PAD_ROWS], W, b.reshape(1, D))
    return out
